# baseline (speedup 1.0000x reference)
import jax
import jax.numpy as jnp
from jax import lax
from jax.experimental import pallas as pl
from jax.experimental.pallas import tpu as pltpu
from jax.experimental.pallas import tpu_sc as plsc

B, L = 4096, 200
N = B * L
VOCAB = 1001
D = 128
PVOCAB = 1008
TE_ROWS = 184
NC, NS, LANES = 2, 16, 16
NW = NC * NS
ROWS_PER_W = N // NW
CHUNK = 128
NCHUNK = ROWS_PER_W // CHUNK
NPAIR = NCHUNK // 2


def _sc_body(codes_hbm, minutes_hbm, values_hbm, ptab_hbm, te_hbm, out_hbm,
             ptab_s, te_s, codes_v, minutes_v, vals_v, stage_v, sems):
    wid = lax.axis_index("s") * NC + lax.axis_index("c")
    lane = lax.iota(jnp.int32, LANES)
    col127 = jnp.full((LANES,), D - 1, jnp.int32)
    isem, gsem, wsem = sems

    @pl.when(lax.axis_index("s") == 0)
    def _stage_tables():
        pltpu.sync_copy(ptab_hbm, ptab_s)
        pltpu.sync_copy(te_hbm, te_s)
    plsc.subcore_barrier()

    def idx_copies(g, x):
        base = wid * ROWS_PER_W + g * CHUNK
        return (
            pltpu.make_async_copy(codes_hbm.at[pl.ds(base, CHUNK)],
                                  codes_v[x], isem[x]),
            pltpu.make_async_copy(minutes_hbm.at[pl.ds(base, CHUNK)],
                                  minutes_v[x], isem[x]),
            pltpu.make_async_copy(values_hbm.at[pl.ds(base, CHUNK)],
                                  vals_v[x], isem[x]),
        )

    def gather_copies(x):
        return (
            pltpu.make_async_copy(ptab_hbm.at[codes_v[x]],
                                  stage_v[x].at[:, pl.ds(0, D)], gsem[x]),
            pltpu.make_async_copy(te_hbm.at[minutes_v[x]],
                                  stage_v[x].at[:, pl.ds(D, D)], gsem[x]),
        )

    def wb_copy(g, x):
        base = wid * ROWS_PER_W + g * CHUNK
        return pltpu.make_async_copy(
            stage_v[x], out_hbm.at[pl.ds(base, CHUNK), :], wsem[x])

    def run_chunk(p, x):
        g = 2 * p + x
        for c in idx_copies(g, x):
            c.wait()
        @pl.when(p >= 1)
        def _():
            wb_copy(g, x).wait()
        for c in gather_copies(x):
            c.start()
        if x == 0:
            for c in idx_copies(g + 1, 1):
                c.start()
        else:
            @pl.when(p <= NPAIR - 2)
            def _():
                for c in idx_copies(g + 1, 0):
                    c.start()
        for c in gather_copies(x):
            c.wait()
        for i in range(CHUNK // LANES):
            v16 = vals_v[x][pl.ds(i * LANES, LANES)]
            row = i * LANES + lane
            plsc.store_scatter(stage_v[x], [row, col127], v16)
        wb_copy(g, x).start()

    for c in idx_copies(0, 0):
        c.start()

    def pair_body(p, _):
        run_chunk(p, 0)
        run_chunk(p, 1)
        return ()

    lax.fori_loop(0, NPAIR, pair_body, ())
    wb_copy(0, 0).wait()
    wb_copy(0, 1).wait()


@jax.jit
def kernel(codes, values, minutes, table, time_encoding):
    codes_f = codes.reshape(N)
    minutes_f = minutes.reshape(N)
    values_f = values.reshape(N)
    ptab = jnp.pad(table, ((0, PVOCAB - VOCAB), (0, 1)))
    te = jnp.pad(time_encoding, ((0, TE_ROWS - 180), (0, 0)))

    mesh = plsc.VectorSubcoreMesh(core_axis_name="c", subcore_axis_name="s",
                                  num_cores=NC, num_subcores=NS)
    out2 = pl.kernel(
        _sc_body,
        out_type=jax.ShapeDtypeStruct((N, 2 * D), jnp.float32),
        mesh=mesh,
        compiler_params=pltpu.CompilerParams(needs_layout_passes=False),
        scratch_types=[
            pltpu.VMEM_SHARED((PVOCAB, D), jnp.float32),
            pltpu.VMEM_SHARED((TE_ROWS, D), jnp.float32),
            [pltpu.VMEM((CHUNK,), jnp.int32)] * 2,
            [pltpu.VMEM((CHUNK,), jnp.int32)] * 2,
            [pltpu.VMEM((CHUNK,), jnp.float32)] * 2,
            [pltpu.VMEM((CHUNK, 2 * D), jnp.float32)] * 2,
            [[pltpu.SemaphoreType.DMA] * 2] * 3,
        ],
    )(codes_f, minutes_f, values_f, ptab, te)
    return out2.reshape(B, L, 2 * D)


# speedup vs baseline: 3.4007x; 3.4007x over previous
import jax
import jax.numpy as jnp
from jax import lax
from jax.experimental import pallas as pl
from jax.experimental.pallas import tpu as pltpu
from jax.experimental.pallas import tpu_sc as plsc

B, L = 4096, 200
N = B * L
VOCAB = 1001
D = 128
PVOCAB = 1008
TE_ROWS = 184
NC, NS, LANES = 2, 16, 16
NW = NC * NS
ROWS_PER_W = N // NW
CHUNK = 128
NCHUNK = ROWS_PER_W // CHUNK
NPAIR = NCHUNK // 2


def _sc_body(codes_hbm, minutes_hbm, values_hbm, ptab_hbm, te_hbm, out_hbm,
             ptab_s, te_s, codes_v, minutes_v, vals_v, cbuf, tbuf, sems):
    wid = lax.axis_index("s") * NC + lax.axis_index("c")
    lane = lax.iota(jnp.int32, LANES)
    col127 = jnp.full((LANES,), D - 1, jnp.int32)
    isem, gsem, wsem = sems

    @pl.when(lax.axis_index("s") == 0)
    def _stage_tables():
        pltpu.sync_copy(ptab_hbm, ptab_s)
        pltpu.sync_copy(te_hbm, te_s)
    plsc.subcore_barrier()

    def idx_copies(g, x):
        base = wid * ROWS_PER_W + g * CHUNK
        return (
            pltpu.make_async_copy(codes_hbm.at[pl.ds(base, CHUNK)],
                                  codes_v[x], isem[x]),
            pltpu.make_async_copy(minutes_hbm.at[pl.ds(base, CHUNK)],
                                  minutes_v[x], isem[x]),
            pltpu.make_async_copy(values_hbm.at[pl.ds(base, CHUNK)],
                                  vals_v[x], isem[x]),
        )

    def gather_copies(x):
        return (
            pltpu.make_async_copy(ptab_s.at[codes_v[x]], cbuf[x], gsem[x]),
            pltpu.make_async_copy(te_s.at[minutes_v[x]], tbuf[x], gsem[x]),
        )

    def wb_copies(g, x):
        base = wid * ROWS_PER_W + g * CHUNK
        return (
            pltpu.make_async_copy(
                cbuf[x], out_hbm.at[pl.ds(base, CHUNK), pl.ds(0, D)],
                wsem[x]),
            pltpu.make_async_copy(
                tbuf[x], out_hbm.at[pl.ds(base, CHUNK), pl.ds(D, D)],
                wsem[x]),
        )

    def run_chunk(p, x):
        g = 2 * p + x
        for c in idx_copies(g, x):
            c.wait()
        @pl.when(p >= 1)
        def _():
            for c in wb_copies(g, x):
                c.wait()
        for c in gather_copies(x):
            c.start()
        if x == 0:
            for c in idx_copies(g + 1, 1):
                c.start()
        else:
            @pl.when(p <= NPAIR - 2)
            def _():
                for c in idx_copies(g + 1, 0):
                    c.start()
        for c in gather_copies(x):
            c.wait()
        for i in range(CHUNK // LANES):
            v16 = vals_v[x][pl.ds(i * LANES, LANES)]
            row = i * LANES + lane
            plsc.store_scatter(cbuf[x], [row, col127], v16)
        for c in wb_copies(g, x):
            c.start()

    for c in idx_copies(0, 0):
        c.start()

    def pair_body(p, _):
        run_chunk(p, 0)
        run_chunk(p, 1)
        return ()

    lax.fori_loop(0, NPAIR, pair_body, ())
    for x in (0, 1):
        for c in wb_copies(0, x):
            c.wait()


@jax.jit
def kernel(codes, values, minutes, table, time_encoding):
    codes_f = codes.reshape(N)
    minutes_f = minutes.reshape(N)
    values_f = values.reshape(N)
    ptab = jnp.pad(table, ((0, PVOCAB - VOCAB), (0, 1)))
    te = jnp.pad(time_encoding, ((0, TE_ROWS - 180), (0, 0)))

    mesh = plsc.VectorSubcoreMesh(core_axis_name="c", subcore_axis_name="s",
                                  num_cores=NC, num_subcores=NS)
    out2 = pl.kernel(
        _sc_body,
        out_type=jax.ShapeDtypeStruct((N, 2 * D), jnp.float32),
        mesh=mesh,
        compiler_params=pltpu.CompilerParams(needs_layout_passes=False),
        scratch_types=[
            pltpu.VMEM_SHARED((PVOCAB, D), jnp.float32),
            pltpu.VMEM_SHARED((TE_ROWS, D), jnp.float32),
            [pltpu.VMEM((CHUNK,), jnp.int32)] * 2,
            [pltpu.VMEM((CHUNK,), jnp.int32)] * 2,
            [pltpu.VMEM((CHUNK,), jnp.float32)] * 2,
            [pltpu.VMEM((CHUNK, D), jnp.float32)] * 2,
            [pltpu.VMEM((CHUNK, D), jnp.float32)] * 2,
            [[pltpu.SemaphoreType.DMA] * 2] * 3,
        ],
    )(codes_f, minutes_f, values_f, ptab, te)
    return out2.reshape(B, L, 2 * D)


# speedup vs baseline: 3.4023x; 1.0005x over previous
import jax
import jax.numpy as jnp
from jax import lax
from jax.experimental import pallas as pl
from jax.experimental.pallas import tpu as pltpu
from jax.experimental.pallas import tpu_sc as plsc

B, L = 4096, 200
N = B * L
VOCAB = 1001
D = 128
PVOCAB = 1008
TE_ROWS = 184
NC, NS, LANES = 2, 16, 16
NW = NC * NS
ROWS_PER_W = N // NW
CHUNK = 128
NCHUNK = ROWS_PER_W // CHUNK
NPAIR = NCHUNK // 2


def _sc_body(codes_hbm, minutes_hbm, values_hbm, ctab_hbm, out_hbm,
             ctab_s, codes_v, minutes_v, vals_v, cbuf, tbuf, sems):
    wid = lax.axis_index("s") * NC + lax.axis_index("c")
    lane = lax.iota(jnp.int32, LANES)
    col127 = jnp.full((LANES,), D - 1, jnp.int32)
    isem, gsem, wsem = sems

    @pl.when(lax.axis_index("s") == 0)
    def _stage_tables():
        pltpu.sync_copy(ctab_hbm, ctab_s)
    plsc.subcore_barrier()

    def idx_copies(g, x):
        base = wid * ROWS_PER_W + g * CHUNK
        return (
            pltpu.make_async_copy(codes_hbm.at[pl.ds(base, CHUNK)],
                                  codes_v[x], isem[x]),
            pltpu.make_async_copy(minutes_hbm.at[pl.ds(base, CHUNK)],
                                  minutes_v[x], isem[x]),
            pltpu.make_async_copy(values_hbm.at[pl.ds(base, CHUNK)],
                                  vals_v[x], isem[x]),
        )

    def gather_copies(x):
        return (
            pltpu.make_async_copy(ctab_s.at[codes_v[x]], cbuf[x], gsem[x]),
            pltpu.make_async_copy(ctab_s.at[minutes_v[x]], tbuf[x], gsem[x]),
        )

    def wb_copies(g, x):
        base = wid * ROWS_PER_W + g * CHUNK
        return (
            pltpu.make_async_copy(
                cbuf[x], out_hbm.at[pl.ds(base, CHUNK), pl.ds(0, D)],
                wsem[x]),
            pltpu.make_async_copy(
                tbuf[x], out_hbm.at[pl.ds(base, CHUNK), pl.ds(D, D)],
                wsem[x]),
        )

    def run_chunk(p, x):
        g = 2 * p + x
        for c in idx_copies(g, x):
            c.wait()
        @pl.when(p >= 1)
        def _():
            for c in wb_copies(g, x):
                c.wait()
        for c in gather_copies(x):
            c.start()
        if x == 0:
            for c in idx_copies(g + 1, 1):
                c.start()
        else:
            @pl.when(p <= NPAIR - 2)
            def _():
                for c in idx_copies(g + 1, 0):
                    c.start()
        for c in gather_copies(x):
            c.wait()
        for i in range(CHUNK // LANES):
            v16 = vals_v[x][pl.ds(i * LANES, LANES)]
            row = i * LANES + lane
            plsc.store_scatter(cbuf[x], [row, col127], v16)
        for c in wb_copies(g, x):
            c.start()

    for c in idx_copies(0, 0):
        c.start()

    def pair_body(p, _):
        run_chunk(p, 0)
        run_chunk(p, 1)
        return ()

    lax.fori_loop(0, NPAIR, pair_body, ())
    for x in (0, 1):
        for c in wb_copies(0, x):
            c.wait()


@jax.jit
def kernel(codes, values, minutes, table, time_encoding):
    codes_f = codes.reshape(N)
    minutes_f = minutes.reshape(N) + VOCAB
    values_f = values.reshape(N)
    ctab = jnp.concatenate(
        [jnp.pad(table, ((0, 0), (0, 1))), time_encoding], axis=0)

    mesh = plsc.VectorSubcoreMesh(core_axis_name="c", subcore_axis_name="s",
                                  num_cores=NC, num_subcores=NS)
    out2 = pl.kernel(
        _sc_body,
        out_type=jax.ShapeDtypeStruct((N, 2 * D), jnp.float32),
        mesh=mesh,
        compiler_params=pltpu.CompilerParams(needs_layout_passes=False),
        scratch_types=[
            pltpu.VMEM_SHARED((VOCAB + 180, D), jnp.float32),
            [pltpu.VMEM((CHUNK,), jnp.int32)] * 2,
            [pltpu.VMEM((CHUNK,), jnp.int32)] * 2,
            [pltpu.VMEM((CHUNK,), jnp.float32)] * 2,
            [pltpu.VMEM((CHUNK, D), jnp.float32)] * 2,
            [pltpu.VMEM((CHUNK, D), jnp.float32)] * 2,
            [[pltpu.SemaphoreType.DMA] * 2] * 3,
        ],
    )(codes_f, minutes_f, values_f, ctab)
    return out2.reshape(B, L, 2 * D)
